# Initial kernel scaffold; baseline (speedup 1.0000x reference)
import jax
import jax.numpy as jnp
from jax.experimental import pallas as pl


def kernel(uv_map, sh_basis_map, tex0, tex1, tex2, tex3):
    raise NotImplementedError("write your pallas kernel here")


# speedup vs baseline: 37.9992x; 37.9992x over previous
import functools

import jax
import jax.numpy as jnp
from jax import lax
from jax.experimental import pallas as pl
from jax.experimental.pallas import tpu as pltpu
from jax.experimental.pallas import tpu_sc as plsc

_SIZES = (1024, 512, 256, 128)
_C = 16
_BC = 128
_IMG_PIX = 512 * 512
_P_TOTAL = 4 * _IMG_PIX


def _build_sc_kernel():
    info = plsc.get_sparse_core_info()
    nc, ns = info.num_cores, info.num_subcores
    nw = nc * ns
    per_w = _P_TOTAL // nw
    n_chunks = per_w // _BC
    w_per_img = _IMG_PIX // per_w
    mesh = plsc.VectorSubcoreMesh(core_axis_name="c", subcore_axis_name="s")

    @functools.partial(
        pl.kernel,
        mesh=mesh,
        compiler_params=pltpu.CompilerParams(
            needs_layout_passes=False, use_tc_tiling_on_sc=False
        ),
        out_type=jax.ShapeDtypeStruct((_P_TOTAL * _C,), jnp.float32),
        scratch_types=[
            pltpu.VMEM((_BC * 2,), jnp.float32),
            pltpu.VMEM((_BC * 9,), jnp.float32),
            pltpu.VMEM((16, _BC), jnp.int32),
            pltpu.VMEM((16, _BC), jnp.float32),
            pltpu.VMEM((16 * _BC, _C), jnp.float32),
            pltpu.VMEM((_C, _BC), jnp.float32),
            pltpu.SemaphoreType.DMA,
        ],
    )
    def sck(uv_hbm, sh_hbm, t0, t1, t2, t3, out_hbm,
            uv_v, sh_v, idx_v, w_v, rows_v, out_v, sem):
        wid = lax.axis_index("s") * nc + lax.axis_index("c")
        n_img = wid // w_per_img
        ibase = (wid % w_per_img) * per_w
        lane = lax.iota(jnp.int32, 16)
        lane2 = lane * 2
        lane9 = lane * 9
        tables = (t0, t1, t2, t3)

        def chunk_body(ci, carry):
            g0 = wid * per_w + ci * _BC
            pltpu.sync_copy(uv_hbm.at[pl.ds(g0 * 2, _BC * 2)], uv_v)

            def agroup(g, c_):
                u = plsc.load_gather(uv_v, [lane2 + g * 32])
                v = plsc.load_gather(uv_v, [lane2 + (g * 32 + 1)])
                sl = pl.ds(g * 16, 16)
                for l in range(4):
                    s = _SIZES[l]
                    c1 = float(s - 1)
                    sx = u * c1
                    sy = c1 - v * c1
                    xi = jnp.minimum(sx.astype(jnp.int32), s - 2)
                    yi = jnp.minimum(sy.astype(jnp.int32), s - 2)
                    fx = sx - xi.astype(jnp.float32)
                    fy = sy - yi.astype(jnp.float32)
                    gx = 1.0 - fx
                    gy = 1.0 - fy
                    b = yi * s + xi
                    idx_v[4 * l + 0, sl] = b
                    idx_v[4 * l + 1, sl] = b + 1
                    idx_v[4 * l + 2, sl] = b + s
                    idx_v[4 * l + 3, sl] = b + (s + 1)
                    w_v[4 * l + 0, sl] = gx * gy
                    w_v[4 * l + 1, sl] = fx * gy
                    w_v[4 * l + 2, sl] = gx * fy
                    w_v[4 * l + 3, sl] = fx * fy
                return c_

            lax.fori_loop(0, _BC // 16, agroup, 0)

            cps = [
                pltpu.async_copy(
                    tables[k // 4].at[idx_v.at[k]],
                    rows_v.at[pl.ds(k * _BC, _BC)],
                    sem,
                )
                for k in range(16)
            ]
            pltpu.sync_copy(sh_hbm.at[pl.ds(g0 * 9, _BC * 9)], sh_v)
            for cp in cps:
                cp.wait()

            def dgroup(g, c_):
                accs = [jnp.zeros((16,), jnp.float32)] * _C
                pix = lane + g * 16
                for k in range(16):
                    w = w_v[k, pl.ds(g * 16, 16)]
                    rvec = pix + k * _BC
                    for c in range(_C):
                        cvec = jnp.full((16,), c, jnp.int32)
                        val = plsc.load_gather(rows_v, [rvec, cvec])
                        accs[c] = accs[c] + w * val
                for c in range(_C):
                    a = accs[c]
                    if 3 <= c < 12:
                        shv = plsc.load_gather(sh_v, [lane9 + (g * 144 + (c - 3))])
                        a = a * shv
                    out_v[c, pl.ds(g * 16, 16)] = a
                return c_

            lax.fori_loop(0, _BC // 16, dgroup, 0)

            for c in range(_C):
                off = n_img * (_C * _IMG_PIX) + c * _IMG_PIX + ibase + ci * _BC
                pltpu.sync_copy(out_v.at[c], out_hbm.at[pl.ds(off, _BC)])
            return carry

        lax.fori_loop(0, n_chunks, chunk_body, 0)

    return sck


_sc_kernel = None


def kernel(uv_map, sh_basis_map, tex0, tex1, tex2, tex3):
    global _sc_kernel
    if _sc_kernel is None:
        _sc_kernel = _build_sc_kernel()
    uv_flat = uv_map.reshape(-1)
    sh_flat = sh_basis_map.reshape(-1)
    t0 = tex0.reshape(_SIZES[0] * _SIZES[0], _C)
    t1 = tex1.reshape(_SIZES[1] * _SIZES[1], _C)
    t2 = tex2.reshape(_SIZES[2] * _SIZES[2], _C)
    t3 = tex3.reshape(_SIZES[3] * _SIZES[3], _C)
    out = _sc_kernel(uv_flat, sh_flat, t0, t1, t2, t3)
    return out.reshape(4, _C, 512, 512)


# speedup vs baseline: 51.3504x; 1.3514x over previous
import functools

import jax
import jax.numpy as jnp
from jax import lax
from jax.experimental import pallas as pl
from jax.experimental.pallas import tpu as pltpu
from jax.experimental.pallas import tpu_sc as plsc

_SIZES = (1024, 512, 256, 128)
_C = 16
_BIG = 512
_SUB = 128
_NSUB = _BIG // _SUB
_IMG_PIX = 512 * 512
_P_TOTAL = 4 * _IMG_PIX


def _build_sc_kernel():
    info = plsc.get_sparse_core_info()
    nc, ns = info.num_cores, info.num_subcores
    nw = nc * ns
    per_w = _P_TOTAL // nw
    n_big = per_w // _BIG
    w_per_img = _IMG_PIX // per_w
    mesh = plsc.VectorSubcoreMesh(core_axis_name="c", subcore_axis_name="s")

    @functools.partial(
        pl.kernel,
        mesh=mesh,
        compiler_params=pltpu.CompilerParams(
            needs_layout_passes=False, use_tc_tiling_on_sc=False
        ),
        out_type=jax.ShapeDtypeStruct((_P_TOTAL * _C,), jnp.float32),
        scratch_types=[
            pltpu.VMEM((_BIG * 2,), jnp.float32),
            pltpu.VMEM((_BIG * 9,), jnp.float32),
            pltpu.VMEM((16, _BIG), jnp.int32),
            pltpu.VMEM((16, _BIG), jnp.float32),
            pltpu.VMEM((2 * 16 * _SUB, _C), jnp.float32),
            pltpu.VMEM((_C, _BIG), jnp.float32),
            pltpu.SemaphoreType.DMA,
            pltpu.SemaphoreType.DMA,
            pltpu.SemaphoreType.DMA,
        ],
    )
    def sck(uv_hbm, sh_hbm, t0, t1, t2, t3, out_hbm,
            uv_v, sh_v, idx_v, w_v, rows_v, out_v, gsem_a, gsem_b, wsem):
        wid = lax.axis_index("s") * nc + lax.axis_index("c")
        n_img = wid // w_per_img
        ibase = (wid % w_per_img) * per_w
        lane = lax.iota(jnp.int32, 16)
        lane2 = lane * 2
        lane9 = lane * 9
        tables = (t0, t1, t2, t3)
        gsems = (gsem_a, gsem_b)

        def agroup(g, c_):
            u = plsc.load_gather(uv_v, [lane2 + g * 32])
            v = plsc.load_gather(uv_v, [lane2 + (g * 32 + 1)])
            sl = pl.ds(g * 16, 16)
            for l in range(4):
                s = _SIZES[l]
                c1 = float(s - 1)
                sx = u * c1
                sy = c1 - v * c1
                xi = jnp.minimum(sx.astype(jnp.int32), s - 2)
                yi = jnp.minimum(sy.astype(jnp.int32), s - 2)
                fx = sx - xi.astype(jnp.float32)
                fy = sy - yi.astype(jnp.float32)
                gx = 1.0 - fx
                gy = 1.0 - fy
                b = yi * s + xi
                idx_v[4 * l + 0, sl] = b
                idx_v[4 * l + 1, sl] = b + 1
                idx_v[4 * l + 2, sl] = b + s
                idx_v[4 * l + 3, sl] = b + (s + 1)
                w_v[4 * l + 0, sl] = gx * gy
                w_v[4 * l + 1, sl] = fx * gy
                w_v[4 * l + 2, sl] = gx * fy
                w_v[4 * l + 3, sl] = fx * fy
            return c_

        def fire_gathers(j):
            slot = j % 2
            return [
                pltpu.async_copy(
                    tables[k // 4].at[idx_v.at[k, pl.ds(j * _SUB, _SUB)]],
                    rows_v.at[pl.ds((slot * 16 + k) * _SUB, _SUB)],
                    gsems[slot],
                )
                for k in range(16)
            ]

        def make_dgroup(j):
            slot = j % 2

            def dgroup(g, c_):
                col = j * _SUB + g * 16
                accs = [jnp.zeros((16,), jnp.float32)] * _C
                pix = lane + g * 16
                for k in range(16):
                    w = w_v[k, pl.ds(col, 16)]
                    rvec = pix + (slot * 16 + k) * _SUB
                    for c in range(_C):
                        cvec = jnp.full((16,), c, jnp.int32)
                        val = plsc.load_gather(rows_v, [rvec, cvec])
                        accs[c] = accs[c] + w * val
                shbase = lane9 + (j * 1152 + g * 144)
                for c in range(_C):
                    a = accs[c]
                    if 3 <= c < 12:
                        a = a * plsc.load_gather(sh_v, [shbase + (c - 3)])
                    out_v[c, pl.ds(col, 16)] = a
                return c_

            return dgroup

        def out_off(b, c):
            return n_img * (_C * _IMG_PIX) + c * _IMG_PIX + ibase + b * _BIG

        def big_body(b, carry):
            g0 = wid * per_w + b * _BIG

            @pl.when(b > 0)
            def _():
                for c in range(_C):
                    pltpu.make_async_copy(
                        out_v.at[c], out_hbm.at[pl.ds(out_off(b, c), _BIG)], wsem
                    ).wait()

            pltpu.sync_copy(uv_hbm.at[pl.ds(g0 * 2, _BIG * 2)], uv_v)
            pltpu.sync_copy(sh_hbm.at[pl.ds(g0 * 9, _BIG * 9)], sh_v)
            lax.fori_loop(0, _BIG // 16, agroup, 0)

            pend = {0: fire_gathers(0), 1: fire_gathers(1)}
            for j in range(_NSUB):
                for cp in pend.pop(j):
                    cp.wait()
                lax.fori_loop(0, _SUB // 16, make_dgroup(j), 0)
                if j + 2 < _NSUB:
                    pend[j + 2] = fire_gathers(j + 2)

            for c in range(_C):
                pltpu.async_copy(
                    out_v.at[c], out_hbm.at[pl.ds(out_off(b, c), _BIG)], wsem
                )
            return carry

        lax.fori_loop(0, n_big, big_body, 0)
        for c in range(_C):
            pltpu.make_async_copy(
                out_v.at[c], out_hbm.at[pl.ds(out_off(n_big - 1, c), _BIG)], wsem
            ).wait()

    return sck


_sc_kernel = None


def kernel(uv_map, sh_basis_map, tex0, tex1, tex2, tex3):
    global _sc_kernel
    if _sc_kernel is None:
        _sc_kernel = _build_sc_kernel()
    uv_flat = uv_map.reshape(-1)
    sh_flat = sh_basis_map.reshape(-1)
    t0 = tex0.reshape(_SIZES[0] * _SIZES[0], _C)
    t1 = tex1.reshape(_SIZES[1] * _SIZES[1], _C)
    t2 = tex2.reshape(_SIZES[2] * _SIZES[2], _C)
    t3 = tex3.reshape(_SIZES[3] * _SIZES[3], _C)
    out = _sc_kernel(uv_flat, sh_flat, t0, t1, t2, t3)
    return out.reshape(4, _C, 512, 512)
